# baseline (speedup 1.0000x reference)
import functools

import jax
import jax.numpy as jnp
from jax import lax
from jax.experimental import pallas as pl
from jax.experimental.pallas import tpu as pltpu
from jax.experimental.pallas import tpu_sc as plsc

_ACTION_DIM = 10
_NUM_BINS = 256
_EMB = 12
_HID = 128
_TOK = _ACTION_DIM * _EMB


def _sc_gather(actions_t, table_flat, n_tokens):
    info = plsc.get_sparse_core_info()
    nc, ns, L = info.num_cores, info.num_subcores, info.num_lanes
    nw = nc * ns
    C = 256
    per_w = n_tokens // nw
    chunks = per_w // C
    mesh = plsc.VectorSubcoreMesh(core_axis_name="c", subcore_axis_name="s")

    @functools.partial(
        pl.kernel,
        mesh=mesh,
        out_type=jax.ShapeDtypeStruct((_TOK, n_tokens), jnp.float32),
        scratch_types=[
            pltpu.VMEM((_ACTION_DIM * _NUM_BINS * _EMB,), jnp.float32),
            pltpu.VMEM((2, _ACTION_DIM, C), jnp.float32),
            pltpu.VMEM((2, _TOK, C), jnp.float32),
            [pltpu.SemaphoreType.DMA] * 2,
            [pltpu.SemaphoreType.DMA] * 2,
        ],
        compiler_params=pltpu.CompilerParams(needs_layout_passes=False),
    )
    def k(actions_hbm, table_hbm, out_hbm, table_v, act_v, tok_v,
          sem_in, sem_out):
        wid = lax.axis_index("s") * nc + lax.axis_index("c")
        base = wid * per_w
        pltpu.sync_copy(table_hbm, table_v)

        def in_slice(ci):
            return actions_hbm.at[:, pl.ds(base + ci * C, C)]

        def out_slice(ci):
            return out_hbm.at[:, pl.ds(base + ci * C, C)]

        def compute(buf, ci):
            @plsc.parallel_loop(0, C // L, unroll=4)
            def group(g):
                off = g * L
                for d in range(_ACTION_DIM):
                    av = act_v[buf, d, pl.ds(off, L)]
                    a = jnp.clip(av, -1.0, 1.0)
                    a = (a + 1.0) * 127.5
                    bins = a.astype(jnp.int32)
                    rowbase = bins * _EMB + d * (_NUM_BINS * _EMB)
                    for w in range(_EMB):
                        val = plsc.load_gather(table_v, [rowbase + w])
                        tok_v[buf, d * _EMB + w, pl.ds(off, L)] = val

        pltpu.async_copy(in_slice(0), act_v.at[0], sem_in[0])

        def chunk_pair(ci, carry):
            for b in range(2):
                cur = ci + b
                pltpu.make_async_copy(in_slice(cur), act_v.at[b],
                                      sem_in[b]).wait()

                @pl.when(cur + 1 < chunks)
                def _():
                    pltpu.async_copy(in_slice(cur + 1), act_v.at[1 - b],
                                     sem_in[1 - b])

                @pl.when(cur >= 2)
                def _():
                    pltpu.make_async_copy(tok_v.at[b], out_slice(cur - 2),
                                          sem_out[b]).wait()

                compute(b, cur)
                pltpu.async_copy(tok_v.at[b], out_slice(cur), sem_out[b])
            return carry

        lax.fori_loop(0, chunks // 2, lambda i, c: chunk_pair(i * 2, c), 0)
        pltpu.make_async_copy(tok_v.at[0], out_slice(chunks - 2),
                              sem_out[0]).wait()
        pltpu.make_async_copy(tok_v.at[1], out_slice(chunks - 1),
                              sem_out[1]).wait()

    return k(actions_t, table_flat)


def _tc_project(tokens_t, w_bf16, b_row):
    n = tokens_t.shape[1]
    BT = 4096

    def mm(tok_ref, w_ref, b_ref, o_ref):
        x = tok_ref[...].astype(jnp.bfloat16)
        acc = lax.dot_general(
            x, w_ref[...], (((0,), (0,)), ((), ())),
            preferred_element_type=jnp.float32,
        )
        o_ref[...] = acc + b_ref[...]

    return pl.pallas_call(
        mm,
        grid=(n // BT,),
        in_specs=[
            pl.BlockSpec((_TOK, BT), lambda i: (0, i)),
            pl.BlockSpec((_TOK, _HID), lambda i: (0, 0)),
            pl.BlockSpec((1, _HID), lambda i: (0, 0)),
        ],
        out_specs=pl.BlockSpec((BT, _HID), lambda i: (i, 0)),
        out_shape=jax.ShapeDtypeStruct((n, _HID), jnp.float32),
    )(tokens_t, w_bf16, b_row)


def kernel(actions, emb_tables, W, b):
    bsz, t, d = actions.shape
    n = bsz * t
    actions_t = actions.reshape(n, d).T
    table_flat = emb_tables.reshape(-1)
    tokens_t = _sc_gather(actions_t, table_flat, n)
    out = _tc_project(tokens_t, W.astype(jnp.bfloat16), b.reshape(1, _HID))
    return out.reshape(bsz, t, _HID)


# speedup vs baseline: 1.5240x; 1.5240x over previous
import functools

import jax
import jax.numpy as jnp
from jax import lax
from jax.experimental import pallas as pl
from jax.experimental.pallas import tpu as pltpu
from jax.experimental.pallas import tpu_sc as plsc

_ACTION_DIM = 10
_NUM_BINS = 256
_EMB = 12
_HID = 128
_TOK = _ACTION_DIM * _EMB


def _sc_gather(actions_t, table_flat, n_tokens):
    info = plsc.get_sparse_core_info()
    nc, ns, L = info.num_cores, info.num_subcores, info.num_lanes
    nw = nc * ns
    C = 256
    per_w = n_tokens // nw
    chunks = per_w // C
    mesh = plsc.VectorSubcoreMesh(core_axis_name="c", subcore_axis_name="s")

    @functools.partial(
        pl.kernel,
        mesh=mesh,
        out_type=jax.ShapeDtypeStruct((_TOK, n_tokens), jnp.float32),
        scratch_types=[
            pltpu.VMEM((_ACTION_DIM * _NUM_BINS * _EMB,), jnp.float32),
            pltpu.VMEM((2, _ACTION_DIM, C), jnp.float32),
            pltpu.VMEM((2, _TOK, C), jnp.float32),
            [pltpu.SemaphoreType.DMA] * 2,
            [pltpu.SemaphoreType.DMA] * 2,
        ],
        compiler_params=pltpu.CompilerParams(needs_layout_passes=False),
    )
    def k(actions_hbm, table_hbm, out_hbm, table_v, act_v, tok_v,
          sem_in, sem_out):
        wid = lax.axis_index("s") * nc + lax.axis_index("c")
        base = wid * per_w
        pltpu.sync_copy(table_hbm, table_v)

        def in_slice(ci):
            return actions_hbm.at[:, pl.ds(base + ci * C, C)]

        def out_slice(ci):
            return out_hbm.at[:, pl.ds(base + ci * C, C)]

        def compute(buf, ci):
            @plsc.parallel_loop(0, C // L, unroll=2)
            def group(g):
                off = g * L
                for d in range(_ACTION_DIM):
                    av = act_v[buf, d, pl.ds(off, L)]
                    a = jnp.clip(av, -1.0, 1.0)
                    a = (a + 1.0) * 127.5
                    bins = a.astype(jnp.int32)
                    rowbase = bins * _EMB + d * (_NUM_BINS * _EMB)
                    for w in range(_EMB):
                        val = plsc.load_gather(table_v, [rowbase + w])
                        tok_v[buf, d * _EMB + w, pl.ds(off, L)] = val

        pltpu.async_copy(in_slice(0), act_v.at[0], sem_in[0])

        def chunk_pair(ci, carry):
            for b in range(2):
                cur = ci + b
                pltpu.make_async_copy(in_slice(cur), act_v.at[b],
                                      sem_in[b]).wait()

                @pl.when(cur + 1 < chunks)
                def _():
                    pltpu.async_copy(in_slice(cur + 1), act_v.at[1 - b],
                                     sem_in[1 - b])

                @pl.when(cur >= 2)
                def _():
                    pltpu.make_async_copy(tok_v.at[b], out_slice(cur - 2),
                                          sem_out[b]).wait()

                compute(b, cur)
                pltpu.async_copy(tok_v.at[b], out_slice(cur), sem_out[b])
            return carry

        lax.fori_loop(0, chunks // 2, lambda i, c: chunk_pair(i * 2, c), 0)
        pltpu.make_async_copy(tok_v.at[0], out_slice(chunks - 2),
                              sem_out[0]).wait()
        pltpu.make_async_copy(tok_v.at[1], out_slice(chunks - 1),
                              sem_out[1]).wait()

    return k(actions_t, table_flat)


def _tc_project(tokens_t, w_bf16, b_row):
    n = tokens_t.shape[1]
    BT = 4096

    def mm(tok_ref, w_ref, b_ref, o_ref):
        x = tok_ref[...].astype(jnp.bfloat16)
        acc = lax.dot_general(
            x, w_ref[...], (((0,), (0,)), ((), ())),
            preferred_element_type=jnp.float32,
        )
        o_ref[...] = acc + b_ref[...]

    return pl.pallas_call(
        mm,
        grid=(n // BT,),
        in_specs=[
            pl.BlockSpec((_TOK, BT), lambda i: (0, i)),
            pl.BlockSpec((_TOK, _HID), lambda i: (0, 0)),
            pl.BlockSpec((1, _HID), lambda i: (0, 0)),
        ],
        out_specs=pl.BlockSpec((BT, _HID), lambda i: (i, 0)),
        out_shape=jax.ShapeDtypeStruct((n, _HID), jnp.float32),
    )(tokens_t, w_bf16, b_row)


def kernel(actions, emb_tables, W, b):
    bsz, t, d = actions.shape
    n = bsz * t
    actions_t = actions.reshape(n, d).T
    table_flat = emb_tables.reshape(-1)
    tokens_t = _sc_gather(actions_t, table_flat, n)
    out = _tc_project(tokens_t, W.astype(jnp.bfloat16), b.reshape(1, _HID))
    return out.reshape(bsz, t, _HID)


# speedup vs baseline: 2.7662x; 1.8151x over previous
import functools

import jax
import jax.numpy as jnp
from jax import lax
from jax.experimental import pallas as pl
from jax.experimental.pallas import tpu as pltpu
from jax.experimental.pallas import tpu_sc as plsc

_ACTION_DIM = 10
_NUM_BINS = 256
_EMB = 12
_HID = 128
_WPE = _EMB // 2
_TOKW = _ACTION_DIM * _WPE


def _sc_gather(actions_t, table_packed, n_tokens):
    info = plsc.get_sparse_core_info()
    nc, ns, L = info.num_cores, info.num_subcores, info.num_lanes
    nw = nc * ns
    C = 512
    per_w = n_tokens // nw
    chunks = per_w // C
    mesh = plsc.VectorSubcoreMesh(core_axis_name="c", subcore_axis_name="s")

    @functools.partial(
        pl.kernel,
        mesh=mesh,
        out_type=jax.ShapeDtypeStruct((_TOKW, n_tokens), jnp.int32),
        scratch_types=[
            pltpu.VMEM((_ACTION_DIM * _NUM_BINS * _WPE,), jnp.int32),
            pltpu.VMEM((2, _ACTION_DIM, C), jnp.float32),
            pltpu.VMEM((2, _TOKW, C), jnp.int32),
            [pltpu.SemaphoreType.DMA] * 2,
            [pltpu.SemaphoreType.DMA] * 2,
        ],
        compiler_params=pltpu.CompilerParams(needs_layout_passes=False),
    )
    def k(actions_hbm, table_hbm, out_hbm, table_v, act_v, tok_v,
          sem_in, sem_out):
        wid = lax.axis_index("s") * nc + lax.axis_index("c")
        base = wid * per_w
        pltpu.sync_copy(table_hbm, table_v)

        def in_slice(ci):
            return actions_hbm.at[:, pl.ds(base + ci * C, C)]

        def out_slice(ci):
            return out_hbm.at[:, pl.ds(base + ci * C, C)]

        def compute(buf):
            @plsc.parallel_loop(0, C // L, unroll=2)
            def group(g):
                off = g * L
                for d in range(_ACTION_DIM):
                    av = act_v[buf, d, pl.ds(off, L)]
                    a = jnp.clip(av, -1.0, 1.0)
                    a = (a + 1.0) * 127.5
                    bins = a.astype(jnp.int32)
                    rowbase = bins * _WPE + d * (_NUM_BINS * _WPE)
                    for w in range(_WPE):
                        val = plsc.load_gather(table_v, [rowbase + w])
                        tok_v[buf, d * _WPE + w, pl.ds(off, L)] = val

        pltpu.async_copy(in_slice(0), act_v.at[0], sem_in[0])

        def chunk_pair(ci, carry):
            for b in range(2):
                cur = ci + b
                pltpu.make_async_copy(in_slice(cur), act_v.at[b],
                                      sem_in[b]).wait()

                @pl.when(cur + 1 < chunks)
                def _():
                    pltpu.async_copy(in_slice(cur + 1), act_v.at[1 - b],
                                     sem_in[1 - b])

                @pl.when(cur >= 2)
                def _():
                    pltpu.make_async_copy(tok_v.at[b], out_slice(cur - 2),
                                          sem_out[b]).wait()

                compute(b)
                pltpu.async_copy(tok_v.at[b], out_slice(cur), sem_out[b])
            return carry

        lax.fori_loop(0, chunks // 2, lambda i, c: chunk_pair(i * 2, c), 0)
        pltpu.make_async_copy(tok_v.at[0], out_slice(chunks - 2),
                              sem_out[0]).wait()
        pltpu.make_async_copy(tok_v.at[1], out_slice(chunks - 1),
                              sem_out[1]).wait()

    return k(actions_t, table_packed)


def _tc_project(tokens_p, w_perm, b_row):
    n = tokens_p.shape[1]
    BT = 4096

    def mm(tok_ref, w_ref, b_ref, o_ref):
        x = tok_ref[...]
        even = lax.bitcast_convert_type(x << 16, jnp.float32)
        odd = lax.bitcast_convert_type((x >> 16) << 16, jnp.float32)
        xx = jnp.concatenate([even, odd], axis=0).astype(jnp.bfloat16)
        acc = lax.dot_general(
            xx, w_ref[...], (((0,), (0,)), ((), ())),
            preferred_element_type=jnp.float32,
        )
        o_ref[...] = acc + b_ref[...]

    return pl.pallas_call(
        mm,
        grid=(n // BT,),
        in_specs=[
            pl.BlockSpec((_TOKW, BT), lambda i: (0, i)),
            pl.BlockSpec((2 * _TOKW, _HID), lambda i: (0, 0)),
            pl.BlockSpec((1, _HID), lambda i: (0, 0)),
        ],
        out_specs=pl.BlockSpec((BT, _HID), lambda i: (i, 0)),
        out_shape=jax.ShapeDtypeStruct((n, _HID), jnp.float32),
    )(tokens_p, w_perm, b_row)


def kernel(actions, emb_tables, W, b):
    bsz, t, d = actions.shape
    n = bsz * t
    actions_t = actions.reshape(n, d).T
    tb = emb_tables.astype(jnp.bfloat16)
    bits = lax.bitcast_convert_type(tb, jnp.uint16).astype(jnp.uint32)
    bits = bits.reshape(_ACTION_DIM, _NUM_BINS, _WPE, 2)
    packed = (bits[..., 0] | (bits[..., 1] << 16)).astype(jnp.int32)
    table_packed = packed.reshape(-1)
    tokens_p = _sc_gather(actions_t, table_packed, n)
    w_perm = jnp.concatenate([W[0::2], W[1::2]], axis=0).astype(jnp.bfloat16)
    out = _tc_project(tokens_p, w_perm, b.reshape(1, _HID))
    return out.reshape(bsz, t, _HID)


# speedup vs baseline: 2.9871x; 1.0798x over previous
import functools

import jax
import jax.numpy as jnp
from jax import lax
from jax.experimental import pallas as pl
from jax.experimental.pallas import tpu as pltpu
from jax.experimental.pallas import tpu_sc as plsc

_ACTION_DIM = 10
_NUM_BINS = 256
_EMB = 12
_HID = 128
_WPE = _EMB // 2
_TOKW = _ACTION_DIM * _WPE


def _sc_gather(actions_t, table_packed, n_tokens):
    info = plsc.get_sparse_core_info()
    nc, ns, L = info.num_cores, info.num_subcores, info.num_lanes
    nw = nc * ns
    C = 512
    per_w = n_tokens // nw
    chunks = per_w // C
    mesh = plsc.VectorSubcoreMesh(core_axis_name="c", subcore_axis_name="s")

    @functools.partial(
        pl.kernel,
        mesh=mesh,
        out_type=jax.ShapeDtypeStruct((_TOKW, n_tokens), jnp.int32),
        scratch_types=[
            pltpu.VMEM((_ACTION_DIM * _NUM_BINS * _WPE,), jnp.int32),
            pltpu.VMEM((2, _ACTION_DIM, C), jnp.float32),
            pltpu.VMEM((2, _TOKW, C), jnp.int32),
            [pltpu.SemaphoreType.DMA] * 2,
            [pltpu.SemaphoreType.DMA] * 2,
        ],
        compiler_params=pltpu.CompilerParams(needs_layout_passes=False),
    )
    def k(actions_hbm, table_hbm, out_hbm, table_v, act_v, tok_v,
          sem_in, sem_out):
        wid = lax.axis_index("s") * nc + lax.axis_index("c")
        base = wid * per_w
        pltpu.sync_copy(table_hbm, table_v)

        def in_slice(ci):
            return actions_hbm.at[:, pl.ds(base + ci * C, C)]

        def out_slice(ci):
            return out_hbm.at[:, pl.ds(base + ci * C, C)]

        def compute(buf):
            @plsc.parallel_loop(0, C // L, unroll=2)
            def group(g):
                off = g * L
                for d in range(_ACTION_DIM):
                    av = act_v[buf, d, pl.ds(off, L)]
                    a = jnp.clip(av, -1.0, 1.0)
                    a = (a + 1.0) * 127.5
                    bins = a.astype(jnp.int32)
                    rowbase = bins * _WPE + d * (_NUM_BINS * _WPE)
                    for w in range(_WPE):
                        val = plsc.load_gather(table_v, [rowbase + w])
                        tok_v[buf, d * _WPE + w, pl.ds(off, L)] = val

        pltpu.async_copy(in_slice(0), act_v.at[0], sem_in[0])

        def chunk_pair(ci, carry):
            for b in range(2):
                cur = ci + b
                pltpu.make_async_copy(in_slice(cur), act_v.at[b],
                                      sem_in[b]).wait()

                @pl.when(cur + 1 < chunks)
                def _():
                    pltpu.async_copy(in_slice(cur + 1), act_v.at[1 - b],
                                     sem_in[1 - b])

                @pl.when(cur >= 2)
                def _():
                    pltpu.make_async_copy(tok_v.at[b], out_slice(cur - 2),
                                          sem_out[b]).wait()

                compute(b)
                pltpu.async_copy(tok_v.at[b], out_slice(cur), sem_out[b])
            return carry

        lax.fori_loop(0, chunks // 2, lambda i, c: chunk_pair(i * 2, c), 0)
        pltpu.make_async_copy(tok_v.at[0], out_slice(chunks - 2),
                              sem_out[0]).wait()
        pltpu.make_async_copy(tok_v.at[1], out_slice(chunks - 1),
                              sem_out[1]).wait()

    return k(actions_t, table_packed)


def _tc_project(tokens_p, w_perm, b_row):
    n = tokens_p.shape[1]
    BT = 8192

    def mm(tok_ref, w_ref, b_ref, o_ref):
        x = tok_ref[...]
        even = lax.bitcast_convert_type(x << 16, jnp.float32)
        odd = lax.bitcast_convert_type((x >> 16) << 16, jnp.float32)
        dn = (((0,), (0,)), ((), ()))
        acc = lax.dot_general(
            even.astype(jnp.bfloat16), w_ref[0:_TOKW, :], dn,
            preferred_element_type=jnp.float32,
        )
        acc += lax.dot_general(
            odd.astype(jnp.bfloat16), w_ref[_TOKW:2 * _TOKW, :], dn,
            preferred_element_type=jnp.float32,
        )
        o_ref[...] = acc + b_ref[...]

    return pl.pallas_call(
        mm,
        grid=(n // BT,),
        in_specs=[
            pl.BlockSpec((_TOKW, BT), lambda i: (0, i)),
            pl.BlockSpec((2 * _TOKW, _HID), lambda i: (0, 0)),
            pl.BlockSpec((1, _HID), lambda i: (0, 0)),
        ],
        out_specs=pl.BlockSpec((BT, _HID), lambda i: (i, 0)),
        out_shape=jax.ShapeDtypeStruct((n, _HID), jnp.float32),
        compiler_params=pltpu.CompilerParams(
            fuse_transposed_lhs_in_matmul=True,
        ),
    )(tokens_p, w_perm, b_row)


def kernel(actions, emb_tables, W, b):
    bsz, t, d = actions.shape
    n = bsz * t
    actions_t = actions.reshape(n, d).T
    tb = emb_tables.astype(jnp.bfloat16)
    bits = lax.bitcast_convert_type(tb, jnp.uint16).astype(jnp.uint32)
    bits = bits.reshape(_ACTION_DIM, _NUM_BINS, _WPE, 2)
    packed = (bits[..., 0] | (bits[..., 1] << 16)).astype(jnp.int32)
    table_packed = packed.reshape(-1)
    tokens_p = _sc_gather(actions_t, table_packed, n)
    w_perm = jnp.concatenate([W[0::2], W[1::2]], axis=0).astype(jnp.bfloat16)
    out = _tc_project(tokens_p, w_perm, b.reshape(1, _HID))
    return out.reshape(bsz, t, _HID)


# speedup vs baseline: 3.0899x; 1.0344x over previous
import functools

import jax
import jax.numpy as jnp
from jax import lax
from jax.experimental import pallas as pl
from jax.experimental.pallas import tpu as pltpu
from jax.experimental.pallas import tpu_sc as plsc

_ACTION_DIM = 10
_NUM_BINS = 256
_EMB = 12
_HID = 128
_WPE = _EMB // 2
_TOKW = _ACTION_DIM * _WPE


def _sc_gather(actions_t, table_packed, n_tokens):
    info = plsc.get_sparse_core_info()
    nc, ns, L = info.num_cores, info.num_subcores, info.num_lanes
    nw = nc * ns
    C = 512
    per_w = n_tokens // nw
    chunks = per_w // C
    mesh = plsc.VectorSubcoreMesh(core_axis_name="c", subcore_axis_name="s")

    @functools.partial(
        pl.kernel,
        mesh=mesh,
        out_type=jax.ShapeDtypeStruct((_TOKW, n_tokens), jnp.int32),
        scratch_types=[
            pltpu.VMEM((_ACTION_DIM * _NUM_BINS * _WPE,), jnp.int32),
            pltpu.VMEM((2, _ACTION_DIM, C), jnp.float32),
            pltpu.VMEM((2, _TOKW, C), jnp.int32),
            [pltpu.SemaphoreType.DMA] * 2,
            [pltpu.SemaphoreType.DMA] * 2,
        ],
        compiler_params=pltpu.CompilerParams(needs_layout_passes=False),
    )
    def k(actions_hbm, table_hbm, out_hbm, table_v, act_v, tok_v,
          sem_in, sem_out):
        wid = lax.axis_index("s") * nc + lax.axis_index("c")
        base = wid * per_w
        pltpu.sync_copy(table_hbm, table_v)

        def in_slice(ci):
            return actions_hbm.at[:, pl.ds(base + ci * C, C)]

        def out_slice(ci):
            return out_hbm.at[:, pl.ds(base + ci * C, C)]

        def compute(buf):
            @plsc.parallel_loop(0, C // L, unroll=2)
            def group(g):
                off = g * L
                for d in range(_ACTION_DIM):
                    av = act_v[buf, d, pl.ds(off, L)]
                    a = jnp.clip(av, -1.0, 1.0)
                    a = (a + 1.0) * 127.5
                    bins = a.astype(jnp.int32)
                    rowbase = bins * _WPE + d * (_NUM_BINS * _WPE)
                    for w in range(_WPE):
                        val = plsc.load_gather(table_v, [rowbase + w])
                        tok_v[buf, d * _WPE + w, pl.ds(off, L)] = val

        pltpu.async_copy(in_slice(0), act_v.at[0], sem_in[0])

        def chunk_pair(ci, carry):
            for b in range(2):
                cur = ci + b
                pltpu.make_async_copy(in_slice(cur), act_v.at[b],
                                      sem_in[b]).wait()

                @pl.when(cur + 1 < chunks)
                def _():
                    pltpu.async_copy(in_slice(cur + 1), act_v.at[1 - b],
                                     sem_in[1 - b])

                @pl.when(cur >= 2)
                def _():
                    pltpu.make_async_copy(tok_v.at[b], out_slice(cur - 2),
                                          sem_out[b]).wait()

                compute(b)
                pltpu.async_copy(tok_v.at[b], out_slice(cur), sem_out[b])
            return carry

        lax.fori_loop(0, chunks // 2, lambda i, c: chunk_pair(i * 2, c), 0)
        pltpu.make_async_copy(tok_v.at[0], out_slice(chunks - 2),
                              sem_out[0]).wait()
        pltpu.make_async_copy(tok_v.at[1], out_slice(chunks - 1),
                              sem_out[1]).wait()

    return k(actions_t, table_packed)


def _tc_project(tokens_p, w_perm, b_row):
    n = tokens_p.shape[1]
    BT = 16384

    def mm(tok_ref, w_ref, b_ref, o_ref):
        x = tok_ref[...]
        even = lax.bitcast_convert_type(x << 16, jnp.float32)
        odd = lax.bitcast_convert_type((x >> 16) << 16, jnp.float32)
        dn = (((0,), (0,)), ((), ()))
        acc = lax.dot_general(
            even.astype(jnp.bfloat16), w_ref[0:_TOKW, :], dn,
            preferred_element_type=jnp.float32,
        )
        acc += lax.dot_general(
            odd.astype(jnp.bfloat16), w_ref[_TOKW:2 * _TOKW, :], dn,
            preferred_element_type=jnp.float32,
        )
        o_ref[...] = acc + b_ref[...]

    return pl.pallas_call(
        mm,
        grid=(n // BT,),
        in_specs=[
            pl.BlockSpec((_TOKW, BT), lambda i: (0, i)),
            pl.BlockSpec((2 * _TOKW, _HID), lambda i: (0, 0)),
            pl.BlockSpec((1, _HID), lambda i: (0, 0)),
        ],
        out_specs=pl.BlockSpec((BT, _HID), lambda i: (i, 0)),
        out_shape=jax.ShapeDtypeStruct((n, _HID), jnp.float32),
        compiler_params=pltpu.CompilerParams(
            fuse_transposed_lhs_in_matmul=True,
        ),
    )(tokens_p, w_perm, b_row)


def kernel(actions, emb_tables, W, b):
    bsz, t, d = actions.shape
    n = bsz * t
    actions_t = actions.reshape(n, d).T
    tb = emb_tables.astype(jnp.bfloat16)
    bits = lax.bitcast_convert_type(tb, jnp.uint16).astype(jnp.uint32)
    bits = bits.reshape(_ACTION_DIM, _NUM_BINS, _WPE, 2)
    packed = (bits[..., 0] | (bits[..., 1] << 16)).astype(jnp.int32)
    table_packed = packed.reshape(-1)
    tokens_p = _sc_gather(actions_t, table_packed, n)
    w_perm = jnp.concatenate([W[0::2], W[1::2]], axis=0).astype(jnp.bfloat16)
    out = _tc_project(tokens_p, w_perm, b.reshape(1, _HID))
    return out.reshape(bsz, t, _HID)


# speedup vs baseline: 3.1187x; 1.0093x over previous
import functools

import jax
import jax.numpy as jnp
from jax import lax
from jax.experimental import pallas as pl
from jax.experimental.pallas import tpu as pltpu
from jax.experimental.pallas import tpu_sc as plsc

_ACTION_DIM = 10
_NUM_BINS = 256
_EMB = 12
_HID = 128
_WPE = _EMB // 2
_TOKW = _ACTION_DIM * _WPE


def _sc_gather(actions_t, table_packed, n_tokens):
    info = plsc.get_sparse_core_info()
    nc, ns, L = info.num_cores, info.num_subcores, info.num_lanes
    nw = nc * ns
    C = 512
    per_w = n_tokens // nw
    chunks = per_w // C
    mesh = plsc.VectorSubcoreMesh(core_axis_name="c", subcore_axis_name="s")

    @functools.partial(
        pl.kernel,
        mesh=mesh,
        out_type=jax.ShapeDtypeStruct((_TOKW, n_tokens), jnp.int32),
        scratch_types=[
            pltpu.VMEM((_ACTION_DIM * _NUM_BINS * _WPE,), jnp.int32),
            pltpu.VMEM((2, _ACTION_DIM, C), jnp.float32),
            pltpu.VMEM((2, _TOKW, C), jnp.int32),
            [pltpu.SemaphoreType.DMA] * 2,
            [pltpu.SemaphoreType.DMA] * 2,
        ],
        compiler_params=pltpu.CompilerParams(needs_layout_passes=False),
    )
    def k(actions_hbm, table_hbm, out_hbm, table_v, act_v, tok_v,
          sem_in, sem_out):
        wid = lax.axis_index("s") * nc + lax.axis_index("c")
        base = wid * per_w
        pltpu.sync_copy(table_hbm, table_v)

        def in_slice(ci):
            return actions_hbm.at[:, pl.ds(base + ci * C, C)]

        def out_slice(ci):
            return out_hbm.at[:, pl.ds(base + ci * C, C)]

        def compute(buf):
            @plsc.parallel_loop(0, C // L, unroll=2)
            def group(g):
                off = g * L
                for d in range(_ACTION_DIM):
                    av = act_v[buf, d, pl.ds(off, L)]
                    a = jnp.clip(av, -1.0, 1.0)
                    a = (a + 1.0) * 127.5
                    bins = a.astype(jnp.int32)
                    rowbase = bins * _WPE + d * (_NUM_BINS * _WPE)
                    for w in range(_WPE):
                        val = plsc.load_gather(table_v, [rowbase + w])
                        tok_v[buf, d * _WPE + w, pl.ds(off, L)] = val

        pltpu.async_copy(in_slice(0), act_v.at[0], sem_in[0])

        def chunk_pair(ci, carry):
            for b in range(2):
                cur = ci + b
                pltpu.make_async_copy(in_slice(cur), act_v.at[b],
                                      sem_in[b]).wait()

                @pl.when(cur + 1 < chunks)
                def _():
                    pltpu.async_copy(in_slice(cur + 1), act_v.at[1 - b],
                                     sem_in[1 - b])

                @pl.when(cur >= 2)
                def _():
                    pltpu.make_async_copy(tok_v.at[b], out_slice(cur - 2),
                                          sem_out[b]).wait()

                compute(b)
                pltpu.async_copy(tok_v.at[b], out_slice(cur), sem_out[b])
            return carry

        lax.fori_loop(0, chunks // 2, lambda i, c: chunk_pair(i * 2, c), 0)
        pltpu.make_async_copy(tok_v.at[0], out_slice(chunks - 2),
                              sem_out[0]).wait()
        pltpu.make_async_copy(tok_v.at[1], out_slice(chunks - 1),
                              sem_out[1]).wait()

    return k(actions_t, table_packed)


def _tc_project(tokens_p, w_perm, b_row):
    n = tokens_p.shape[1]
    BT = 32768

    def mm(tok_ref, w_ref, b_ref, o_ref):
        x = tok_ref[...]
        even = lax.bitcast_convert_type(x << 16, jnp.float32)
        odd = lax.bitcast_convert_type((x >> 16) << 16, jnp.float32)
        dn = (((0,), (0,)), ((), ()))
        acc = lax.dot_general(
            even.astype(jnp.bfloat16), w_ref[0:_TOKW, :], dn,
            preferred_element_type=jnp.float32,
        )
        acc += lax.dot_general(
            odd.astype(jnp.bfloat16), w_ref[_TOKW:2 * _TOKW, :], dn,
            preferred_element_type=jnp.float32,
        )
        o_ref[...] = acc + b_ref[...]

    return pl.pallas_call(
        mm,
        grid=(n // BT,),
        in_specs=[
            pl.BlockSpec((_TOKW, BT), lambda i: (0, i)),
            pl.BlockSpec((2 * _TOKW, _HID), lambda i: (0, 0)),
            pl.BlockSpec((1, _HID), lambda i: (0, 0)),
        ],
        out_specs=pl.BlockSpec((BT, _HID), lambda i: (i, 0)),
        out_shape=jax.ShapeDtypeStruct((n, _HID), jnp.float32),
        compiler_params=pltpu.CompilerParams(
            fuse_transposed_lhs_in_matmul=True,
        ),
    )(tokens_p, w_perm, b_row)


def kernel(actions, emb_tables, W, b):
    bsz, t, d = actions.shape
    n = bsz * t
    actions_t = actions.reshape(n, d).T
    tb = emb_tables.astype(jnp.bfloat16)
    bits = lax.bitcast_convert_type(tb, jnp.uint16).astype(jnp.uint32)
    bits = bits.reshape(_ACTION_DIM, _NUM_BINS, _WPE, 2)
    packed = (bits[..., 0] | (bits[..., 1] << 16)).astype(jnp.int32)
    table_packed = packed.reshape(-1)
    tokens_p = _sc_gather(actions_t, table_packed, n)
    w_perm = jnp.concatenate([W[0::2], W[1::2]], axis=0).astype(jnp.bfloat16)
    out = _tc_project(tokens_p, w_perm, b.reshape(1, _HID))
    return out.reshape(bsz, t, _HID)
